# baseline (speedup 1.0000x reference)
import functools

import jax
import jax.numpy as jnp
from jax import lax
from jax.experimental import pallas as pl
from jax.experimental.pallas import tpu as pltpu
from jax.experimental.pallas import tpu_sc as plsc

NS = 16
L = 16
CH = 32


@functools.lru_cache(maxsize=None)
def _make_sc(N, V, B, T):
    RPW = N // NS
    VB = V // 128
    NCH = RPW // CH
    mesh = plsc.VectorSubcoreMesh(
        core_axis_name="c", subcore_axis_name="s", num_cores=1)

    @functools.partial(
        pl.kernel,
        out_type=jax.ShapeDtypeStruct((L,), jnp.float32),
        mesh=mesh,
        compiler_params=pltpu.CompilerParams(
            needs_layout_passes=False,
            skip_device_barrier=True,
            disable_bounds_checks=True,
        ),
        scratch_types=[
            pltpu.VMEM((RPW,), jnp.int32),
            pltpu.VMEM((RPW,), jnp.float32),
            pltpu.VMEM((RPW,), jnp.int32),
            pltpu.VMEM((RPW,), jnp.int32),
            pltpu.VMEM((RPW, 128), jnp.float32),
            pltpu.VMEM((128,), jnp.float32),
            pltpu.VMEM((NS, 128), jnp.float32),
            pltpu.VMEM((L,), jnp.float32),
            pltpu.VMEM_SHARED((NS, 128), jnp.float32),
            pltpu.SemaphoreType.DMA,
            pltpu.SemaphoreType.DMA,
        ] + [pltpu.SemaphoreType.DMA] * (RPW // CH),
    )
    def k(in_hbm, tgt_hbm, msk_hbm, out_hbm,
          tgt_v, msk_v, idx_v, col_v, rows_v, acc_v, all_v, o_v, shared,
          sem, sem2, *gsems):
        sid = lax.axis_index("s")
        base = sid * RPW
        b = base // T
        blk = (base % T) // 128
        NB = RPW // 128
        flat = in_hbm.reshape(N * V // 128, 128)
        for i in range(NB):
            pltpu.async_copy(tgt_hbm.at[blk + i, b],
                             tgt_v.at[pl.ds(i * 128, 128)], sem)
            pltpu.async_copy(msk_hbm.at[blk + i, b],
                             msk_v.at[pl.ds(i * 128, 128)], sem2)
        lane = lax.iota(jnp.int32, L)
        zero16 = jnp.zeros((L,), jnp.int32)

        def zero_body(j, _):
            idx_v[pl.ds(pl.multiple_of(j * L, L), L)] = zero16
            return 0

        lax.fori_loop(0, RPW // L, zero_body, 0)
        for i in range(NB):
            pltpu.make_async_copy(tgt_hbm.at[blk, b],
                                  tgt_v.at[pl.ds(0, 128)], sem).wait()
            pltpu.make_async_copy(msk_hbm.at[blk, b],
                                  msk_v.at[pl.ds(0, 128)], sem2).wait()

        def compact_body(j, o):
            t = tgt_v[pl.ds(pl.multiple_of(j * L, L), L)]
            m = msk_v[pl.ds(pl.multiple_of(j * L, L), L)]
            n = (base + j * L) + lane
            q = (lax.shift_right_logical(n, 3) * (VB * 8)
                 + lax.shift_right_logical(t, 7) * 8
                 + jnp.bitwise_and(n, 7))
            valid = m != 0.0
            plsc.store_compressed(idx_v.at[pl.ds(o, L)], q, mask=valid)
            plsc.store_compressed(col_v.at[pl.ds(o, L)],
                                  jnp.bitwise_and(t, 127), mask=valid)
            pc = plsc.all_reduce_population_count(valid)
            return o + lax.reduce_max_p.bind(pc, axes=(0,))

        o = lax.fori_loop(0, RPW // L, compact_body, jnp.int32(0))

        gathers = []
        for c in range(NCH):
            @pl.when(c * CH < o)
            def _(c=c):
                gathers.append(pltpu.async_copy(
                    flat.at[idx_v.at[pl.ds(c * CH, CH)]],
                    rows_v.at[pl.ds(c * CH, CH)], gsems[c]))
        acc_v[pl.ds(0, L)] = jnp.zeros((L,), jnp.float32)
        for c in range(NCH):
            @pl.when(c * CH < o)
            def _(c=c):
                gathers[c].wait()
                part = jnp.zeros((L,), jnp.float32)
                for jj in range(CH // L):
                    j = c * (CH // L) + jj
                    pos = j * L + lane
                    col = jnp.bitwise_and(col_v[pl.ds(j * L, L)], 127)
                    v = plsc.load_gather(rows_v, [pos, col])
                    part = part + jnp.where(pos < o, v, 0.0)
                acc_v[pl.ds(0, L)] = acc_v[pl.ds(0, L)] - part
        o_f = lax.broadcast_in_dim(
            lax.convert_element_type(o, jnp.float32), (L,), ())
        acc_v[pl.ds(L, L)] = jnp.where(lane == 0, o_f, 0.0)
        pltpu.sync_copy(acc_v, shared.at[sid])
        plsc.subcore_barrier()

        @pl.when(sid == 0)
        def _():
            pltpu.sync_copy(shared, all_v)

            def red_body(i, carry):
                s, c = carry
                return (s + all_v[i, pl.ds(0, L)],
                        c + all_v[i, pl.ds(L, L)])

            s, c = lax.fori_loop(
                0, NS, red_body,
                (jnp.zeros((L,), jnp.float32), jnp.zeros((L,), jnp.float32)))
            S = lax.broadcast_in_dim(
                lax.reduce_sum_p.bind(s, axes=(0,)), (L,), ())
            C = lax.broadcast_in_dim(
                lax.reduce_sum_p.bind(c, axes=(0,)), (L,), ())
            o_v[...] = S / C
            pltpu.sync_copy(o_v, out_hbm)

    return k


def kernel(input, target, mask):
    B, T, V = input.shape
    target = target[:, :T]
    mask = mask[:, :T]
    N = B * T
    x5 = input.reshape(B, T // 8, 8, V // 128, 128).transpose(0, 1, 3, 2, 4)
    tgt = target.astype(jnp.int32).reshape(B, T // 128, 128).transpose(1, 0, 2)
    msk = mask.astype(jnp.float32).reshape(B, T // 128, 128).transpose(1, 0, 2)
    out = _make_sc(N, V, B, T)(x5, tgt, msk)
    return out[0]
